# baseline (speedup 1.0000x reference)
import jax
import jax.numpy as jnp
from jax import lax
from jax.experimental import pallas as pl
from jax.experimental.pallas import tpu as pltpu
from jax.experimental.pallas import tpu_sc as plsc

M = 100000
B = 16384
D = 512
NC, NS = 2, 16
W = NC * NS
CB = B // W
GC = 32
NG = CB // GC
NBUF = 3

BLK = 4000
GRID = M // BLK


def _copy_body(xb, ox):
    ox[...] = xb[...]


def _int_copy_body(yb, tb, ib, sb, oy, ot, oi, osp):
    oy[pl.ds(0, M)] = yb[...]
    ot[pl.ds(0, M)] = tb[...]
    oi[pl.ds(0, M)] = ib[...]
    osp[pl.ds(0, M)] = sb[...]


def _sc_body(bx, by, bt, bidx, bstep, in_x, in_y, in_t, in_bidx, in_step, swap,
             obx, oby, obt, obidx, obstep,
             idxa, idxf, g0, g1, g2, r0, r1, r2,
             gy, gt, gi, gs, vy, vt, vi, vs,
             sga, sgb, ssa, ssb, semg, semv, sems):
    w = lax.axis_index("s") * NC + lax.axis_index("c")
    jlo = pl.multiple_of(w * CB, 8)

    for c in range(NG):
        pltpu.sync_copy(swap.at[pl.ds(jlo + c * GC, GC)], idxa.at[c])
    pltpu.sync_copy(swap.at[pl.ds(jlo, CB)], idxf)

    lv = [
        pltpu.async_copy(in_y.at[pl.ds(jlo, CB)], vy, semv),
        pltpu.async_copy(in_t.at[pl.ds(jlo, CB)], vt, semv),
        pltpu.async_copy(in_bidx.at[pl.ds(jlo, CB)], vi, semv),
        pltpu.async_copy(in_step.at[pl.ds(jlo, CB)], vs, semv),
    ]
    ig = [
        pltpu.async_copy(by.at[idxf], gy, semg),
        pltpu.async_copy(bt.at[idxf], gt, semg),
        pltpu.async_copy(bidx.at[idxf], gi, semg),
        pltpu.async_copy(bstep.at[idxf], gs, semg),
    ]

    grows = (g0, g1, g2)
    srows = (r0, r1, r2)
    gl = [None, None, None]
    gst = [None, None, None]
    sl = [None, None, None]
    sst = [None, None, None]
    for c in range(2):
        gl[c] = pltpu.async_copy(bx.at[idxa.at[c]], grows[c], sga)
        sl[c] = pltpu.async_copy(in_x.at[pl.ds(jlo + c * GC, GC)],
                                 srows[c], ssa)
    for c in range(NG):
        b = c % NBUF
        nx = (c + 2) % NBUF
        gl[b].wait()
        sl[b].wait()
        if c + 2 < NG:
            if gst[nx] is not None:
                gst[nx].wait()
                sst[nx].wait()
            gl[nx] = pltpu.async_copy(bx.at[idxa.at[c + 2]], grows[nx], sga)
            sl[nx] = pltpu.async_copy(in_x.at[pl.ds(jlo + (c + 2) * GC, GC)],
                                      srows[nx], ssa)
        gst[b] = pltpu.async_copy(grows[b],
                                  obx.at[pl.ds(M + jlo + c * GC, GC)], sgb)
        sst[b] = pltpu.async_copy(srows[b], obx.at[idxa.at[c]], ssb)
    for b in range(NBUF):
        gst[b].wait()
        sst[b].wait()

    for cp in ig:
        cp.wait()
    pltpu.sync_copy(gy, oby.at[pl.ds(M + jlo, CB)])
    pltpu.sync_copy(gt, obt.at[pl.ds(M + jlo, CB)])
    pltpu.sync_copy(gi, obidx.at[pl.ds(M + jlo, CB)])
    pltpu.sync_copy(gs, obstep.at[pl.ds(M + jlo, CB)])

    for cp in lv:
        cp.wait()
    isc = [
        pltpu.async_copy(vy, oby.at[idxf], sems),
        pltpu.async_copy(vt, obt.at[idxf], sems),
        pltpu.async_copy(vi, obidx.at[idxf], sems),
        pltpu.async_copy(vs, obstep.at[idxf], sems),
    ]
    for cp in isc:
        cp.wait()


def kernel(bx, by, bt, bidx, bstep, in_x, in_y, in_t, in_bidx, in_step, swap_idx):
    copy_kernel = pl.pallas_call(
        _copy_body,
        grid=(GRID,),
        in_specs=[pl.BlockSpec((BLK, D), lambda i: (i, 0))],
        out_specs=pl.BlockSpec((BLK, D), lambda i: (i, 0)),
        out_shape=jax.ShapeDtypeStruct((M + B, D), jnp.float32),
        compiler_params=pltpu.CompilerParams(
            dimension_semantics=("parallel",)),
    )
    int_copy_kernel = pl.pallas_call(
        _int_copy_body,
        out_shape=(
            jax.ShapeDtypeStruct((M + B,), jnp.int32),
            jax.ShapeDtypeStruct((M + B,), jnp.int32),
            jax.ShapeDtypeStruct((M + B,), jnp.int32),
            jax.ShapeDtypeStruct((M + B,), jnp.int32),
        ),
    )

    mesh = plsc.VectorSubcoreMesh(core_axis_name="c", subcore_axis_name="s")
    fsem = pltpu.SemaphoreType.DMA
    sc_kernel = pl.kernel(
        _sc_body,
        out_type=(),
        mesh=mesh,
        scratch_types=[
            pltpu.VMEM((NG, GC), jnp.int32),
            pltpu.VMEM((CB,), jnp.int32),
            pltpu.VMEM((GC, D), jnp.float32),
            pltpu.VMEM((GC, D), jnp.float32),
            pltpu.VMEM((GC, D), jnp.float32),
            pltpu.VMEM((GC, D), jnp.float32),
            pltpu.VMEM((GC, D), jnp.float32),
            pltpu.VMEM((GC, D), jnp.float32),
            pltpu.VMEM((CB,), jnp.int32),
            pltpu.VMEM((CB,), jnp.int32),
            pltpu.VMEM((CB,), jnp.int32),
            pltpu.VMEM((CB,), jnp.int32),
            pltpu.VMEM((CB,), jnp.int32),
            pltpu.VMEM((CB,), jnp.int32),
            pltpu.VMEM((CB,), jnp.int32),
            pltpu.VMEM((CB,), jnp.int32),
            fsem, fsem, fsem, fsem, fsem, fsem, fsem,
        ],
    )

    ox = copy_kernel(bx)
    oints = int_copy_kernel(by, bt, bidx, bstep)
    refs = [jax.new_ref(o) for o in (ox, *oints)]
    sc_kernel(bx, by, bt, bidx, bstep,
              in_x, in_y, in_t, in_bidx, in_step, swap_idx, *refs)
    return tuple(r[...] for r in refs)


# speedup vs baseline: 1.0077x; 1.0077x over previous
import jax
import jax.numpy as jnp
from jax import lax
from jax.experimental import pallas as pl
from jax.experimental.pallas import tpu as pltpu
from jax.experimental.pallas import tpu_sc as plsc

M = 100000
B = 16384
D = 512
NC, NS = 2, 16
W = NC * NS
CB = B // W
GC = 32
NG = CB // GC
NBUF = 3

BLK = 4000
GRID = M // BLK
IBLK = 4096


def _copy_body(xb, yb, tb, ib, sb, ox, oy, ot, oi, osp):
    ox[...] = xb[...]
    oy[...] = yb[...]
    ot[...] = tb[...]
    oi[...] = ib[...]
    osp[...] = sb[...]


def _sc_body(bx, by, bt, bidx, bstep, in_x, in_y, in_t, in_bidx, in_step, swap,
             obx, oby, obt, obidx, obstep,
             idxa, idxf, g0, g1, g2, r0, r1, r2,
             gy, gt, gi, gs, vy, vt, vi, vs,
             sga, sgb, ssa, ssb, semg, semv, sems):
    w = lax.axis_index("s") * NC + lax.axis_index("c")
    jlo = pl.multiple_of(w * CB, 8)

    for c in range(NG):
        pltpu.sync_copy(swap.at[pl.ds(jlo + c * GC, GC)], idxa.at[c])
    pltpu.sync_copy(swap.at[pl.ds(jlo, CB)], idxf)

    lv = [
        pltpu.async_copy(in_y.at[pl.ds(jlo, CB)], vy, semv),
        pltpu.async_copy(in_t.at[pl.ds(jlo, CB)], vt, semv),
        pltpu.async_copy(in_bidx.at[pl.ds(jlo, CB)], vi, semv),
        pltpu.async_copy(in_step.at[pl.ds(jlo, CB)], vs, semv),
    ]
    ig = [
        pltpu.async_copy(by.at[idxf], gy, semg),
        pltpu.async_copy(bt.at[idxf], gt, semg),
        pltpu.async_copy(bidx.at[idxf], gi, semg),
        pltpu.async_copy(bstep.at[idxf], gs, semg),
    ]

    grows = (g0, g1, g2)
    srows = (r0, r1, r2)
    gl = [None, None, None]
    gst = [None, None, None]
    sl = [None, None, None]
    sst = [None, None, None]
    for c in range(2):
        gl[c] = pltpu.async_copy(bx.at[idxa.at[c]], grows[c], sga)
        sl[c] = pltpu.async_copy(in_x.at[pl.ds(jlo + c * GC, GC)],
                                 srows[c], ssa)
    for c in range(NG):
        b = c % NBUF
        nx = (c + 2) % NBUF
        gl[b].wait()
        sl[b].wait()
        if c + 2 < NG:
            if gst[nx] is not None:
                gst[nx].wait()
                sst[nx].wait()
            gl[nx] = pltpu.async_copy(bx.at[idxa.at[c + 2]], grows[nx], sga)
            sl[nx] = pltpu.async_copy(in_x.at[pl.ds(jlo + (c + 2) * GC, GC)],
                                      srows[nx], ssa)
        gst[b] = pltpu.async_copy(grows[b],
                                  obx.at[pl.ds(M + jlo + c * GC, GC)], sgb)
        sst[b] = pltpu.async_copy(srows[b], obx.at[idxa.at[c]], ssb)
    for b in range(NBUF):
        gst[b].wait()
        sst[b].wait()

    for cp in ig:
        cp.wait()
    pltpu.sync_copy(gy, oby.at[pl.ds(M + jlo, CB)])
    pltpu.sync_copy(gt, obt.at[pl.ds(M + jlo, CB)])
    pltpu.sync_copy(gi, obidx.at[pl.ds(M + jlo, CB)])
    pltpu.sync_copy(gs, obstep.at[pl.ds(M + jlo, CB)])

    for cp in lv:
        cp.wait()
    isc = [
        pltpu.async_copy(vy, oby.at[idxf], sems),
        pltpu.async_copy(vt, obt.at[idxf], sems),
        pltpu.async_copy(vi, obidx.at[idxf], sems),
        pltpu.async_copy(vs, obstep.at[idxf], sems),
    ]
    for cp in isc:
        cp.wait()


def kernel(bx, by, bt, bidx, bstep, in_x, in_y, in_t, in_bidx, in_step, swap_idx):
    ispec = pl.BlockSpec((IBLK,), lambda i: (i,))
    copy_kernel = pl.pallas_call(
        _copy_body,
        grid=(GRID,),
        in_specs=[pl.BlockSpec((BLK, D), lambda i: (i, 0)),
                  ispec, ispec, ispec, ispec],
        out_specs=[pl.BlockSpec((BLK, D), lambda i: (i, 0)),
                   ispec, ispec, ispec, ispec],
        out_shape=(
            jax.ShapeDtypeStruct((M + B, D), jnp.float32),
            jax.ShapeDtypeStruct((M + B,), jnp.int32),
            jax.ShapeDtypeStruct((M + B,), jnp.int32),
            jax.ShapeDtypeStruct((M + B,), jnp.int32),
            jax.ShapeDtypeStruct((M + B,), jnp.int32),
        ),
        compiler_params=pltpu.CompilerParams(
            dimension_semantics=("parallel",)),
    )

    mesh = plsc.VectorSubcoreMesh(core_axis_name="c", subcore_axis_name="s")
    fsem = pltpu.SemaphoreType.DMA
    sc_kernel = pl.kernel(
        _sc_body,
        out_type=(),
        mesh=mesh,
        scratch_types=[
            pltpu.VMEM((NG, GC), jnp.int32),
            pltpu.VMEM((CB,), jnp.int32),
            pltpu.VMEM((GC, D), jnp.float32),
            pltpu.VMEM((GC, D), jnp.float32),
            pltpu.VMEM((GC, D), jnp.float32),
            pltpu.VMEM((GC, D), jnp.float32),
            pltpu.VMEM((GC, D), jnp.float32),
            pltpu.VMEM((GC, D), jnp.float32),
            pltpu.VMEM((CB,), jnp.int32),
            pltpu.VMEM((CB,), jnp.int32),
            pltpu.VMEM((CB,), jnp.int32),
            pltpu.VMEM((CB,), jnp.int32),
            pltpu.VMEM((CB,), jnp.int32),
            pltpu.VMEM((CB,), jnp.int32),
            pltpu.VMEM((CB,), jnp.int32),
            pltpu.VMEM((CB,), jnp.int32),
            fsem, fsem, fsem, fsem, fsem, fsem, fsem,
        ],
    )

    outs = copy_kernel(bx, by, bt, bidx, bstep)
    refs = [jax.new_ref(o) for o in outs]
    sc_kernel(bx, by, bt, bidx, bstep,
              in_x, in_y, in_t, in_bidx, in_step, swap_idx, *refs)
    return tuple(r[...] for r in refs)


# speedup vs baseline: 1.0330x; 1.0251x over previous
import jax
import jax.numpy as jnp
from jax import lax
from jax.experimental import pallas as pl
from jax.experimental.pallas import tpu as pltpu
from jax.experimental.pallas import tpu_sc as plsc

M = 100000
B = 16384
D = 512
NC, NS = 2, 16
W = NC * NS
CB = B // W
GC = 32
NG = CB // GC
NBUF = 3

BLK = 4000
GRID = M // BLK
IBLK = 4096


def _copy_body(xb, yb, tb, ib, sb, ox, oy, ot, oi, osp):
    ox[...] = xb[...]
    oy[...] = yb[...]
    ot[...] = tb[...]
    oi[...] = ib[...]
    osp[...] = sb[...]


def _sc_body(bx, by, bt, bidx, bstep, in_x, in_y, in_t, in_bidx, in_step, swap,
             obx, oby, obt, obidx, obstep,
             idxa, idxf, g0, g1, g2, r0, r1, r2,
             gy, gt, gi, gs, vy, vt, vi, vs,
             sga, sgb, ssa, ssb, semg, semv, sems):
    w = lax.axis_index("s") * NC + lax.axis_index("c")
    jlo = pl.multiple_of(w * CB, 8)

    del idxa
    pltpu.sync_copy(swap.at[pl.ds(jlo, CB)], idxf)

    def idx(c):
        return idxf.at[pl.ds(c * GC, GC)]

    lv = [
        pltpu.async_copy(in_y.at[pl.ds(jlo, CB)], vy, semv),
        pltpu.async_copy(in_t.at[pl.ds(jlo, CB)], vt, semv),
        pltpu.async_copy(in_bidx.at[pl.ds(jlo, CB)], vi, semv),
        pltpu.async_copy(in_step.at[pl.ds(jlo, CB)], vs, semv),
    ]
    ig = [
        pltpu.async_copy(by.at[idxf], gy, semg),
        pltpu.async_copy(bt.at[idxf], gt, semg),
        pltpu.async_copy(bidx.at[idxf], gi, semg),
        pltpu.async_copy(bstep.at[idxf], gs, semg),
    ]

    grows = (g0, g1, g2)
    srows = (r0, r1, r2)
    gl = [None, None, None]
    gst = [None, None, None]
    sl = [None, None, None]
    sst = [None, None, None]
    for c in range(2):
        gl[c] = pltpu.async_copy(bx.at[idx(c)], grows[c], sga)
        sl[c] = pltpu.async_copy(in_x.at[pl.ds(jlo + c * GC, GC)],
                                 srows[c], ssa)
    for c in range(NG):
        b = c % NBUF
        nx = (c + 2) % NBUF
        gl[b].wait()
        sl[b].wait()
        if c + 2 < NG:
            if gst[nx] is not None:
                gst[nx].wait()
                sst[nx].wait()
            gl[nx] = pltpu.async_copy(bx.at[idx(c + 2)], grows[nx], sga)
            sl[nx] = pltpu.async_copy(in_x.at[pl.ds(jlo + (c + 2) * GC, GC)],
                                      srows[nx], ssa)
        gst[b] = pltpu.async_copy(grows[b],
                                  obx.at[pl.ds(M + jlo + c * GC, GC)], sgb)
        sst[b] = pltpu.async_copy(srows[b], obx.at[idx(c)], ssb)
    for b in range(NBUF):
        gst[b].wait()
        sst[b].wait()

    for cp in ig:
        cp.wait()
    pltpu.sync_copy(gy, oby.at[pl.ds(M + jlo, CB)])
    pltpu.sync_copy(gt, obt.at[pl.ds(M + jlo, CB)])
    pltpu.sync_copy(gi, obidx.at[pl.ds(M + jlo, CB)])
    pltpu.sync_copy(gs, obstep.at[pl.ds(M + jlo, CB)])

    for cp in lv:
        cp.wait()
    isc = [
        pltpu.async_copy(vy, oby.at[idxf], sems),
        pltpu.async_copy(vt, obt.at[idxf], sems),
        pltpu.async_copy(vi, obidx.at[idxf], sems),
        pltpu.async_copy(vs, obstep.at[idxf], sems),
    ]
    for cp in isc:
        cp.wait()


def kernel(bx, by, bt, bidx, bstep, in_x, in_y, in_t, in_bidx, in_step, swap_idx):
    ispec = pl.BlockSpec((IBLK,), lambda i: (i,))
    copy_kernel = pl.pallas_call(
        _copy_body,
        grid=(GRID,),
        in_specs=[pl.BlockSpec((BLK, D), lambda i: (i, 0)),
                  ispec, ispec, ispec, ispec],
        out_specs=[pl.BlockSpec((BLK, D), lambda i: (i, 0)),
                   ispec, ispec, ispec, ispec],
        out_shape=(
            jax.ShapeDtypeStruct((M + B, D), jnp.float32),
            jax.ShapeDtypeStruct((M + B,), jnp.int32),
            jax.ShapeDtypeStruct((M + B,), jnp.int32),
            jax.ShapeDtypeStruct((M + B,), jnp.int32),
            jax.ShapeDtypeStruct((M + B,), jnp.int32),
        ),
        compiler_params=pltpu.CompilerParams(
            dimension_semantics=("parallel",)),
    )

    mesh = plsc.VectorSubcoreMesh(core_axis_name="c", subcore_axis_name="s")
    fsem = pltpu.SemaphoreType.DMA
    sc_kernel = pl.kernel(
        _sc_body,
        out_type=(),
        mesh=mesh,
        scratch_types=[
            pltpu.VMEM((NG, GC), jnp.int32),
            pltpu.VMEM((CB,), jnp.int32),
            pltpu.VMEM((GC, D), jnp.float32),
            pltpu.VMEM((GC, D), jnp.float32),
            pltpu.VMEM((GC, D), jnp.float32),
            pltpu.VMEM((GC, D), jnp.float32),
            pltpu.VMEM((GC, D), jnp.float32),
            pltpu.VMEM((GC, D), jnp.float32),
            pltpu.VMEM((CB,), jnp.int32),
            pltpu.VMEM((CB,), jnp.int32),
            pltpu.VMEM((CB,), jnp.int32),
            pltpu.VMEM((CB,), jnp.int32),
            pltpu.VMEM((CB,), jnp.int32),
            pltpu.VMEM((CB,), jnp.int32),
            pltpu.VMEM((CB,), jnp.int32),
            pltpu.VMEM((CB,), jnp.int32),
            fsem, fsem, fsem, fsem, fsem, fsem, fsem,
        ],
    )

    outs = copy_kernel(bx, by, bt, bidx, bstep)
    refs = [jax.new_ref(o) for o in outs]
    sc_kernel(bx, by, bt, bidx, bstep,
              in_x, in_y, in_t, in_bidx, in_step, swap_idx, *refs)
    return tuple(r[...] for r in refs)
